# baseline (speedup 1.0000x reference)
import functools

import jax
import jax.numpy as jnp
from jax import lax
from jax.experimental import pallas as pl
from jax.experimental.pallas import tpu as pltpu
from jax.experimental.pallas import tpu_sc as plsc

NC = 2
NS = 16
LANES = 16
CHUNK = 128
NBUF = 4
G = 8


def _spmm_sc(x_cols, pdata, wdata, n_groups, n_pad, d2, bits):
    rows_per_tile = n_pad // NS
    zrep = rows_per_tile // CHUNK
    assert rows_per_tile % CHUNK == 0
    n_chunks = n_groups * G
    mask = (1 << bits) - 1
    mesh = plsc.VectorSubcoreMesh(core_axis_name="c", subcore_axis_name="s")

    @functools.partial(
        pl.kernel,
        out_type=jax.ShapeDtypeStruct((NC, n_pad, d2), jnp.float32),
        mesh=mesh,
        scratch_types=[
            [pltpu.VMEM((G, CHUNK), jnp.int32) for _ in range(2)],
            [pltpu.VMEM((G, CHUNK), jnp.float32) for _ in range(2)],
            [pltpu.VMEM((2, CHUNK), jnp.int32) for _ in range(NBUF)],
            [pltpu.VMEM((CHUNK, d2), jnp.float32) for _ in range(NBUF)],
            pltpu.VMEM_SHARED((n_pad, d2), jnp.float32),
            pltpu.VMEM_SHARED((n_pad, d2), jnp.float32),
            [pltpu.SemaphoreType.DMA for _ in range(2 + 2 * NBUF)],
        ],
        compiler_params=pltpu.CompilerParams(use_tc_tiling_on_sc=False),
    )
    def spmm(x_hbm, p_hbm, w_hbm, out_hbm, pst, wst, ibuf, rbuf, x_sp,
             acc, sems):
        c = lax.axis_index("c")
        s = lax.axis_index("s")
        base_r = s * rows_per_tile
        rows = pl.ds(base_r, rows_per_tile)
        esem = sems[0:2]
        gsem = sems[2:2 + NBUF]
        ssem = sems[2 + NBUF:2 + 2 * NBUF]

        def zrow(i, carry):
            for t in range(d2 // LANES):
                rbuf[0][i, pl.ds(t * LANES, LANES)] = jnp.zeros(
                    (LANES,), jnp.float32)
            return carry

        lax.fori_loop(0, CHUNK, zrow, 0, unroll=False)
        for z in range(zrep):
            pltpu.sync_copy(
                rbuf[0], acc.at[pl.ds(base_r + z * CHUNK, CHUNK)])
        pltpu.sync_copy(x_hbm.at[c, rows], x_sp.at[rows])
        plsc.subcore_barrier()

        def start_E(g, b):
            pltpu.async_copy(p_hbm.at[s, g], pst[b], esem[b])
            pltpu.async_copy(w_hbm.at[s, g], wst[b], esem[b])

        def wait_E(b):
            pltpu.make_async_copy(p_hbm.at[s, 0], pst[b], esem[b]).wait()
            pltpu.make_async_copy(w_hbm.at[s, 0], wst[b], esem[b]).wait()

        def unpack(k, b, p):
            for t in range(CHUNK // LANES):
                sl = pl.ds(t * LANES, LANES)
                pk = pst[b][k, sl]
                ibuf[p][0, sl] = pk & mask
                ibuf[p][1, sl] = lax.shift_right_logical(pk, bits)

        def start_g(p):
            pltpu.async_copy(x_sp.at[ibuf[p].at[0]], rbuf[p], gsem[p])

        def wait_g(p):
            pltpu.make_async_copy(x_sp.at[ibuf[p].at[0]], rbuf[p],
                                  gsem[p]).wait()

        def start_s(p):
            pltpu.async_copy(rbuf[p], acc.at[ibuf[p].at[1]], ssem[p],
                             add=True)

        def wait_s(p):
            pltpu.make_async_copy(rbuf[p], acc.at[ibuf[p].at[1]],
                                  ssem[p]).wait()

        def scale(k, b, p):
            def grp_body(g2, carry2):
                w16 = wst[b][k, pl.ds(g2 * LANES, LANES)]
                for r in range(LANES):
                    i = g2 * LANES + r
                    wv = w16[r]
                    for t in range(d2 // LANES):
                        sl = pl.ds(t * LANES, LANES)
                        rbuf[p][i, sl] = rbuf[p][i, sl] * wv
                return carry2

            lax.fori_loop(0, CHUNK // LANES, grp_body, 0, unroll=False)

        start_E(0, 0)
        start_E(1, 1)
        wait_E(0)
        unpack(0, 0, 0)
        start_g(0)

        def iter_body(mm, carry):
            for u in range(2 * G):
                gg, k = divmod(u, G)
                ph = u % NBUF
                j = 2 * G * mm + u
                g = 2 * mm + gg

                @pl.when((j >= 2) & (j + 2 < n_chunks))
                def _(ph=ph):
                    wait_s((ph + 2) % NBUF)

                if k == 0:
                    @pl.when((g >= 1) & (g + 1 < n_groups))
                    def _(gg=gg):
                        start_E(g + 1, (gg + 1) % 2)

                @pl.when(j + 1 < n_chunks)
                def _(ph=ph, gg=gg, k=k, g=g):
                    q1 = (ph + 1) % NBUF
                    if k == G - 1:
                        wait_E((gg + 1) % 2)
                        unpack(0, (gg + 1) % 2, q1)
                    else:
                        unpack(k + 1, gg, q1)
                    start_g(q1)

                wait_g(ph)
                scale(k, gg, ph)
                start_s(ph)
            return carry

        lax.fori_loop(0, n_chunks // (2 * G), iter_body, 0, unroll=False)
        for p in range(NBUF):
            wait_s(p)
        plsc.subcore_barrier()

        pltpu.sync_copy(acc.at[rows], out_hbm.at[c, rows])

    return spmm(x_cols, pdata, wdata)


def _matmul_tc(partials, W):
    _, n, d2 = partials.shape
    bn = 512
    assert n % bn == 0

    def body(p_ref, w_ref, o_ref):
        o_ref[...] = (
            jnp.dot(p_ref[0], w_ref[:d2, :],
                    preferred_element_type=jnp.float32)
            + jnp.dot(p_ref[1], w_ref[d2:, :],
                      preferred_element_type=jnp.float32))

    return pl.pallas_call(
        body,
        grid=(n // bn,),
        in_specs=[
            pl.BlockSpec((NC, bn, d2), lambda i: (0, i, 0)),
            pl.BlockSpec((2 * d2, 2 * d2), lambda i: (0, 0)),
        ],
        out_specs=pl.BlockSpec((bn, 2 * d2), lambda i: (i, 0)),
        out_shape=jax.ShapeDtypeStruct((n, 2 * d2), jnp.float32),
    )(partials, W)


def kernel(x, edge_index, edge_weight, W):
    n, d = x.shape
    e = edge_weight.shape[0]
    d2 = d // 2
    bits = max((n - 1).bit_length(), 1)
    assert 2 * bits <= 30, "src/dst must pack into one i32"
    n_pad = -(-n // 1024) * 1024

    per_group = NS * CHUNK * G
    n_groups = -(-e // per_group)
    n_groups = max(n_groups, 2)
    n_chunks = n_groups * G
    e_pad = NS * n_chunks * CHUNK
    src = edge_index[0]
    dst = edge_index[1]
    packed = src | (dst << bits)
    pdata = (jnp.zeros((e_pad,), jnp.int32).at[:e].set(packed)
             .reshape(NS, n_groups, G, CHUNK))
    wdata = (jnp.zeros((e_pad,), jnp.float32).at[:e].set(edge_weight)
             .reshape(NS, n_groups, G, CHUNK))
    x_pad = jnp.zeros((n_pad, d), jnp.float32).at[:n].set(x)
    x_cols = x_pad.reshape(n_pad, NC, d2).transpose(1, 0, 2)

    partials = _spmm_sc(x_cols, pdata, wdata, n_groups, n_pad, d2, bits)
    return _matmul_tc(partials, W)[:n]
